# baseline (speedup 1.0000x reference)
import functools

import jax
import jax.numpy as jnp
from jax import lax
from jax.experimental import pallas as pl
from jax.experimental.pallas import tpu as pltpu
from jax.experimental.pallas import tpu_sc as plsc

VOCAB = 100000
DIM = 64
MAXLEN = 200
BATCH = 1024

NC = 2
NS = 16
NW = NC * NS
ROWS_PER_W = BATCH // NW

IDX_SPLIT = 2
IDX_CHUNK = MAXLEN // IDX_SPLIT


def _make_kernel():
  mesh = plsc.VectorSubcoreMesh(core_axis_name="c", subcore_axis_name="s")

  @functools.partial(
      pl.kernel,
      out_type=jax.ShapeDtypeStruct((BATCH, MAXLEN, DIM), jnp.float32),
      mesh=mesh,
      scratch_types=[
          pltpu.VMEM((IDX_SPLIT, IDX_CHUNK), jnp.int32),
          pltpu.VMEM((MAXLEN, DIM), jnp.float32),
          pltpu.SemaphoreType.DMA,
      ],
      compiler_params=pltpu.CompilerParams(use_tc_tiling_on_sc=False),
  )
  def tok_pos_embed(idx_hbm, tok_hbm, pos_hbm, out_hbm, idx_v, row_v, sem):
    wid = lax.axis_index("s") * NC + lax.axis_index("c")

    def body(r, carry):
      row = wid * ROWS_PER_W + r
      pltpu.sync_copy(idx_hbm.at[row], idx_v)
      pltpu.sync_copy(pos_hbm, row_v)
      cps = [
          pltpu.async_copy(
              tok_hbm.at[idx_v.at[j]],
              row_v.at[pl.ds(j * IDX_CHUNK, IDX_CHUNK)],
              sem,
              add=True,
          )
          for j in range(IDX_SPLIT)
      ]
      for cp in cps:
        cp.wait()
      pltpu.sync_copy(row_v, out_hbm.at[row])
      return carry

    lax.fori_loop(0, ROWS_PER_W, body, 0)

  return tok_pos_embed


_KERNEL = _make_kernel()


def kernel(inputs, token_table, pos_table):
  idx = inputs.astype(jnp.int32).reshape(BATCH, IDX_SPLIT, IDX_CHUNK)
  return _KERNEL(idx, token_table, pos_table)


# speedup vs baseline: 2.1117x; 2.1117x over previous
import functools

import jax
import jax.numpy as jnp
from jax import lax
from jax.experimental import pallas as pl
from jax.experimental.pallas import tpu as pltpu
from jax.experimental.pallas import tpu_sc as plsc

VOCAB = 100000
DIM = 64
PDIM = 128
MAXLEN = 200
BATCH = 1024
LANES = 16

NC = 2
NS = 16
NW = NC * NS
ROWS_PER_W = BATCH // NW

IDX_SPLIT = 2
IDX_CHUNK = MAXLEN // IDX_SPLIT

SEED_UNROLL = 4


def _make_kernel():
  mesh = plsc.VectorSubcoreMesh(core_axis_name="c", subcore_axis_name="s")

  @functools.partial(
      pl.kernel,
      out_type=jax.ShapeDtypeStruct((BATCH, MAXLEN, PDIM), jnp.float32),
      mesh=mesh,
      scratch_types=[
          pltpu.VMEM((ROWS_PER_W, IDX_SPLIT, IDX_CHUNK), jnp.int32),
          pltpu.VMEM((MAXLEN, DIM), jnp.float32),
          pltpu.VMEM((MAXLEN, DIM), jnp.float32),
          pltpu.VMEM((MAXLEN, DIM), jnp.float32),
          pltpu.SemaphoreType.DMA,
          pltpu.SemaphoreType.DMA,
          pltpu.SemaphoreType.DMA,
          pltpu.SemaphoreType.DMA,
      ],
      compiler_params=pltpu.CompilerParams(use_tc_tiling_on_sc=False),
  )
  def tok_pos_embed(idx_hbm, tok_hbm, pos_hbm, out_hbm, idx_all, pos_v,
                    buf0, buf1, g0, g1, o0, o1):
    wid = lax.axis_index("s") * NC + lax.axis_index("c")
    row0 = wid * ROWS_PER_W
    bufs = (buf0, buf1)
    gsem = (g0, g1)
    osem = (o0, o1)

    pltpu.sync_copy(idx_hbm.at[pl.ds(row0, ROWS_PER_W)], idx_all)
    pltpu.sync_copy(pos_hbm, pos_v)

    def seed(buf):
      def sbody(i, carry):
        base = i * SEED_UNROLL
        for u in range(SEED_UNROLL):
          for t in range(DIM // LANES):
            buf[base + u, pl.ds(t * LANES, LANES)] = (
                pos_v[base + u, pl.ds(t * LANES, LANES)])
        return carry
      lax.fori_loop(0, MAXLEN // SEED_UNROLL, sbody, 0)

    def fire(p, r):
      return [
          pltpu.async_copy(
              tok_hbm.at[idx_all.at[r, j]],
              bufs[p].at[pl.ds(j * IDX_CHUNK, IDX_CHUNK)],
              gsem[p],
              add=True,
          )
          for j in range(IDX_SPLIT)
      ]

    def wait_gathers(p, r):
      for j in range(IDX_SPLIT):
        pltpu.make_async_copy(
            tok_hbm.at[idx_all.at[r, j]],
            bufs[p].at[pl.ds(j * IDX_CHUNK, IDX_CHUNK)],
            gsem[p],
        ).wait()

    def fire_out(p, r):
      return pltpu.async_copy(
          bufs[p], out_hbm.at[row0 + r].at[:, pl.ds(0, DIM)], osem[p])

    def wait_out(p, r):
      pltpu.make_async_copy(
          bufs[p], out_hbm.at[row0 + r].at[:, pl.ds(0, DIM)], osem[p]).wait()

    seed(buf0)
    fire(0, 0)
    seed(buf1)
    fire(1, 1)

    def body(g, carry):
      for p in range(2):
        r = 2 * g + p
        wait_gathers(p, r)
        fire_out(p, r)
        nr = r + 2

        @pl.when(nr < ROWS_PER_W)
        def _():
          wait_out(p, r)
          seed(bufs[p])
          fire(p, nr)
      return carry

    lax.fori_loop(0, ROWS_PER_W // 2, body, 0)
    wait_out(0, ROWS_PER_W - 2)
    wait_out(1, ROWS_PER_W - 1)

  return tok_pos_embed


_KERNEL = _make_kernel()


def kernel(inputs, token_table, pos_table):
  idx = inputs.astype(jnp.int32).reshape(BATCH, IDX_SPLIT, IDX_CHUNK)
  out = _KERNEL(idx, token_table, pos_table)
  return out[:, :, :DIM]


# speedup vs baseline: 2.1127x; 1.0004x over previous
import functools

import jax
import jax.numpy as jnp
from jax import lax
from jax.experimental import pallas as pl
from jax.experimental.pallas import tpu as pltpu
from jax.experimental.pallas import tpu_sc as plsc

VOCAB = 100000
DIM = 64
PDIM = 128
MAXLEN = 200
BATCH = 1024
LANES = 16

NC = 2
NS = 16
NW = NC * NS
ROWS_PER_W = BATCH // NW

IDX_SPLIT = 2
IDX_CHUNK = MAXLEN // IDX_SPLIT

SEED_UNROLL = 4


def _make_kernel():
  mesh = plsc.VectorSubcoreMesh(core_axis_name="c", subcore_axis_name="s")

  @functools.partial(
      pl.kernel,
      out_type=jax.ShapeDtypeStruct((BATCH, MAXLEN, PDIM), jnp.float32),
      mesh=mesh,
      scratch_types=[
          pltpu.VMEM((ROWS_PER_W, IDX_SPLIT, IDX_CHUNK), jnp.int32),
          pltpu.VMEM((MAXLEN, DIM), jnp.float32),
          pltpu.VMEM((MAXLEN, DIM), jnp.float32),
          pltpu.VMEM((MAXLEN, DIM), jnp.float32),
          pltpu.SemaphoreType.DMA,
          pltpu.SemaphoreType.DMA,
          pltpu.SemaphoreType.DMA,
          pltpu.SemaphoreType.DMA,
      ],
      compiler_params=pltpu.CompilerParams(
          use_tc_tiling_on_sc=False, needs_layout_passes=True),
  )
  def tok_pos_embed(idx_hbm, tok_hbm, pos_hbm, out_hbm, idx_all, pos_v,
                    buf0, buf1, g0, g1, o0, o1):
    wid = lax.axis_index("s") * NC + lax.axis_index("c")
    row0 = wid * ROWS_PER_W
    bufs = (buf0, buf1)
    gsem = (g0, g1)
    osem = (o0, o1)

    pltpu.sync_copy(idx_hbm.at[pl.ds(row0, ROWS_PER_W)], idx_all)
    pltpu.sync_copy(pos_hbm, pos_v)

    def seed(buf):
      def sbody(i, carry):
        base = i * SEED_UNROLL
        for u in range(SEED_UNROLL):
          for t in range(DIM // LANES):
            buf[base + u, pl.ds(t * LANES, LANES)] = (
                pos_v[base + u, pl.ds(t * LANES, LANES)])
        return carry
      lax.fori_loop(0, MAXLEN // SEED_UNROLL, sbody, 0)

    def fire(p, r):
      return [
          pltpu.async_copy(
              tok_hbm.at[idx_all.at[r, j]],
              bufs[p].at[pl.ds(j * IDX_CHUNK, IDX_CHUNK)],
              gsem[p],
              add=True,
          )
          for j in range(IDX_SPLIT)
      ]

    def wait_gathers(p, r):
      for j in range(IDX_SPLIT):
        pltpu.make_async_copy(
            tok_hbm.at[idx_all.at[r, j]],
            bufs[p].at[pl.ds(j * IDX_CHUNK, IDX_CHUNK)],
            gsem[p],
        ).wait()

    def fire_out(p, r):
      return pltpu.async_copy(
          bufs[p], out_hbm.at[row0 + r].at[:, pl.ds(0, DIM)], osem[p])

    def wait_out(p, r):
      pltpu.make_async_copy(
          bufs[p], out_hbm.at[row0 + r].at[:, pl.ds(0, DIM)], osem[p]).wait()

    seed(buf0)
    fire(0, 0)
    seed(buf1)
    fire(1, 1)

    def body(g, carry):
      for p in range(2):
        r = 2 * g + p
        wait_gathers(p, r)
        fire_out(p, r)
        nr = r + 2

        @pl.when(nr < ROWS_PER_W)
        def _():
          wait_out(p, r)
          seed(bufs[p])
          fire(p, nr)
      return carry

    lax.fori_loop(0, ROWS_PER_W // 2, body, 0)
    wait_out(0, ROWS_PER_W - 2)
    wait_out(1, ROWS_PER_W - 1)

  return tok_pos_embed


_KERNEL = _make_kernel()


def kernel(inputs, token_table, pos_table):
  idx = inputs.astype(jnp.int32).reshape(BATCH, IDX_SPLIT, IDX_CHUNK)
  out = _KERNEL(idx, token_table, pos_table)
  return out[:, :, :DIM]


# speedup vs baseline: 2.1199x; 1.0034x over previous
import functools

import jax
import jax.numpy as jnp
from jax import lax
from jax.experimental import pallas as pl
from jax.experimental.pallas import tpu as pltpu
from jax.experimental.pallas import tpu_sc as plsc

VOCAB = 100000
DIM = 64
PDIM = 128
MAXLEN = 200
BATCH = 1024
LANES = 16

NC = 2
NS = 16
NW = NC * NS
ROWS_PER_W = BATCH // NW

IDX_CHUNKS = (104, 96)
IDX_OFFS = (0, 104)

SEED_UNROLL = 4


def _make_kernel():
  mesh = plsc.VectorSubcoreMesh(core_axis_name="c", subcore_axis_name="s")

  @functools.partial(
      pl.kernel,
      out_type=jax.ShapeDtypeStruct((BATCH, MAXLEN, PDIM), jnp.float32),
      mesh=mesh,
      scratch_types=[
          pltpu.VMEM((ROWS_PER_W * MAXLEN,), jnp.int32),
          pltpu.VMEM((MAXLEN, DIM), jnp.float32),
          pltpu.VMEM((MAXLEN, DIM), jnp.float32),
          pltpu.VMEM((MAXLEN, DIM), jnp.float32),
          pltpu.SemaphoreType.DMA,
          pltpu.SemaphoreType.DMA,
          pltpu.SemaphoreType.DMA,
          pltpu.SemaphoreType.DMA,
      ],
      compiler_params=pltpu.CompilerParams(use_tc_tiling_on_sc=False),
  )
  def tok_pos_embed(idx_hbm, tok_hbm, pos_hbm, out_hbm, idx_all, pos_v,
                    buf0, buf1, g0, g1, o0, o1):
    wid = lax.axis_index("s") * NC + lax.axis_index("c")
    row0 = wid * ROWS_PER_W
    bufs = (buf0, buf1)
    gsem = (g0, g1)
    osem = (o0, o1)

    pltpu.sync_copy(
        idx_hbm.at[pl.ds(row0 * MAXLEN, ROWS_PER_W * MAXLEN)], idx_all)
    pltpu.sync_copy(pos_hbm, pos_v)

    def seed(buf):
      def sbody(i, carry):
        base = i * SEED_UNROLL
        for u in range(SEED_UNROLL):
          for t in range(DIM // LANES):
            buf[base + u, pl.ds(t * LANES, LANES)] = (
                pos_v[base + u, pl.ds(t * LANES, LANES)])
        return carry
      lax.fori_loop(0, MAXLEN // SEED_UNROLL, sbody, 0)

    def fire(p, r):
      return [
          pltpu.async_copy(
              tok_hbm.at[idx_all.at[pl.ds(r * MAXLEN + o, n)]],
              bufs[p].at[pl.ds(o, n)],
              gsem[p],
              add=True,
          )
          for o, n in zip(IDX_OFFS, IDX_CHUNKS)
      ]

    def wait_gathers(p, r):
      for o, n in zip(IDX_OFFS, IDX_CHUNKS):
        pltpu.make_async_copy(
            tok_hbm.at[idx_all.at[pl.ds(r * MAXLEN + o, n)]],
            bufs[p].at[pl.ds(o, n)],
            gsem[p],
        ).wait()

    def fire_out(p, r):
      return pltpu.async_copy(
          bufs[p], out_hbm.at[row0 + r].at[:, pl.ds(0, DIM)], osem[p])

    def wait_out(p, r):
      pltpu.make_async_copy(
          bufs[p], out_hbm.at[row0 + r].at[:, pl.ds(0, DIM)], osem[p]).wait()

    seed(buf0)
    fire(0, 0)
    seed(buf1)
    fire(1, 1)

    def body(g, carry):
      for p in range(2):
        r = 2 * g + p
        wait_gathers(p, r)
        fire_out(p, r)
        nr = r + 2

        @pl.when(nr < ROWS_PER_W)
        def _():
          wait_out(p, r)
          seed(bufs[p])
          fire(p, nr)
      return carry

    lax.fori_loop(0, ROWS_PER_W // 2, body, 0)
    wait_out(0, ROWS_PER_W - 2)
    wait_out(1, ROWS_PER_W - 1)

  return tok_pos_embed


_KERNEL = _make_kernel()


def kernel(inputs, token_table, pos_table):
  idx = inputs.astype(jnp.int32).reshape(BATCH * MAXLEN)
  out = _KERNEL(idx, token_table, pos_table)
  return out[:, :, :DIM]


# speedup vs baseline: 2.1620x; 1.0198x over previous
import functools

import jax
import jax.numpy as jnp
from jax import lax
from jax.experimental import pallas as pl
from jax.experimental.pallas import tpu as pltpu
from jax.experimental.pallas import tpu_sc as plsc

VOCAB = 100000
DIM = 64
PDIM = 128
MAXLEN = 200
BATCH = 1024
LANES = 16

NC = 2
NS = 16
NW = NC * NS
ROWS_PER_W = BATCH // NW

IDX_CHUNKS = (104, 96)
IDX_OFFS = (0, 104)

SEED_UNROLL = 4
NBUF = 4


def _make_kernel():
  mesh = plsc.VectorSubcoreMesh(core_axis_name="c", subcore_axis_name="s")

  @functools.partial(
      pl.kernel,
      out_type=jax.ShapeDtypeStruct((BATCH, MAXLEN, PDIM), jnp.float32),
      mesh=mesh,
      scratch_types=[
          pltpu.VMEM((ROWS_PER_W * MAXLEN,), jnp.int32),
          pltpu.VMEM((MAXLEN, DIM), jnp.float32),
          pltpu.VMEM((MAXLEN, DIM), jnp.float32),
          pltpu.VMEM((MAXLEN, DIM), jnp.float32),
          pltpu.VMEM((MAXLEN, DIM), jnp.float32),
          pltpu.VMEM((MAXLEN, DIM), jnp.float32),
          pltpu.SemaphoreType.DMA,
          pltpu.SemaphoreType.DMA,
          pltpu.SemaphoreType.DMA,
          pltpu.SemaphoreType.DMA,
          pltpu.SemaphoreType.DMA,
          pltpu.SemaphoreType.DMA,
          pltpu.SemaphoreType.DMA,
          pltpu.SemaphoreType.DMA,
      ],
      compiler_params=pltpu.CompilerParams(use_tc_tiling_on_sc=False),
  )
  def tok_pos_embed(idx_hbm, tok_hbm, pos_hbm, out_hbm, idx_all, pos_v,
                    *bufs_and_sems):
    bufs = bufs_and_sems[:NBUF]
    gsem = bufs_and_sems[NBUF:2 * NBUF]
    osem = bufs_and_sems[2 * NBUF:3 * NBUF]
    wid = lax.axis_index("s") * NC + lax.axis_index("c")
    row0 = wid * ROWS_PER_W

    pltpu.sync_copy(
        idx_hbm.at[pl.ds(row0 * MAXLEN, ROWS_PER_W * MAXLEN)], idx_all)
    pltpu.sync_copy(pos_hbm, pos_v)

    def seed(buf):
      def sbody(i, carry):
        base = i * SEED_UNROLL
        for u in range(SEED_UNROLL):
          for t in range(DIM // LANES):
            buf[base + u, pl.ds(t * LANES, LANES)] = (
                pos_v[base + u, pl.ds(t * LANES, LANES)])
        return carry
      lax.fori_loop(0, MAXLEN // SEED_UNROLL, sbody, 0)

    def fire(p, r):
      return [
          pltpu.async_copy(
              tok_hbm.at[idx_all.at[pl.ds(r * MAXLEN + o, n)]],
              bufs[p].at[pl.ds(o, n)],
              gsem[p],
              add=True,
          )
          for o, n in zip(IDX_OFFS, IDX_CHUNKS)
      ]

    def wait_gathers(p, r):
      for o, n in zip(IDX_OFFS, IDX_CHUNKS):
        pltpu.make_async_copy(
            tok_hbm.at[idx_all.at[pl.ds(r * MAXLEN + o, n)]],
            bufs[p].at[pl.ds(o, n)],
            gsem[p],
        ).wait()

    def fire_out(p, r):
      return pltpu.async_copy(
          bufs[p], out_hbm.at[row0 + r].at[:, pl.ds(0, DIM)], osem[p])

    def wait_out(p, r):
      pltpu.make_async_copy(
          bufs[p], out_hbm.at[row0 + r].at[:, pl.ds(0, DIM)], osem[p]).wait()

    for p in range(NBUF):
      seed(bufs[p])
      fire(p, p)

    def body(g, carry):
      for p in range(NBUF):
        r = NBUF * g + p
        wait_gathers(p, r)
        fire_out(p, r)
        nr = r + NBUF

        @pl.when(nr < ROWS_PER_W)
        def _():
          wait_out(p, r)
          seed(bufs[p])
          fire(p, nr)
      return carry

    lax.fori_loop(0, ROWS_PER_W // NBUF, body, 0)
    for p in range(NBUF):
      wait_out(p, ROWS_PER_W - NBUF + p)

  return tok_pos_embed


_KERNEL = _make_kernel()


def kernel(inputs, token_table, pos_table):
  idx = inputs.astype(jnp.int32).reshape(BATCH * MAXLEN)
  out = _KERNEL(idx, token_table, pos_table)
  return out[:, :, :DIM]


# speedup vs baseline: 2.3633x; 1.0931x over previous
import functools

import jax
import jax.numpy as jnp
from jax import lax
from jax.experimental import pallas as pl
from jax.experimental.pallas import tpu as pltpu
from jax.experimental.pallas import tpu_sc as plsc

VOCAB = 100000
DIM = 64
PDIM = 128
MAXLEN = 200
BATCH = 1024
LANES = 16

NC = 2
NS = 16
NW = NC * NS
ROWS_PER_W = BATCH // NW

IDX_CHUNKS = (104, 96)
IDX_OFFS = (0, 104)

SEED_UNROLL = 4
NBUF = 4


def _make_kernel():
  mesh = plsc.VectorSubcoreMesh(core_axis_name="c", subcore_axis_name="s")

  @functools.partial(
      pl.kernel,
      out_type=jax.ShapeDtypeStruct((BATCH, MAXLEN, PDIM), jnp.float32),
      mesh=mesh,
      scratch_types=[
          pltpu.VMEM((ROWS_PER_W * MAXLEN,), jnp.int32),
          pltpu.VMEM((MAXLEN, DIM), jnp.float32),
          pltpu.VMEM((MAXLEN, DIM), jnp.float32),
          pltpu.VMEM((MAXLEN, DIM), jnp.float32),
          pltpu.VMEM((MAXLEN, DIM), jnp.float32),
          pltpu.VMEM((MAXLEN, DIM), jnp.float32),
          pltpu.SemaphoreType.DMA,
          pltpu.SemaphoreType.DMA,
          pltpu.SemaphoreType.DMA,
          pltpu.SemaphoreType.DMA,
          pltpu.SemaphoreType.DMA,
          pltpu.SemaphoreType.DMA,
          pltpu.SemaphoreType.DMA,
          pltpu.SemaphoreType.DMA,
      ],
      compiler_params=pltpu.CompilerParams(use_tc_tiling_on_sc=False),
  )
  def tok_pos_embed(idx_hbm, tok_hbm, pos_hbm, out_hbm, idx_all, pos_v,
                    *bufs_and_sems):
    bufs = bufs_and_sems[:NBUF]
    gsem = bufs_and_sems[NBUF:2 * NBUF]
    osem = bufs_and_sems[2 * NBUF:3 * NBUF]
    wid = lax.axis_index("s") * NC + lax.axis_index("c")
    row0 = wid * ROWS_PER_W

    pltpu.sync_copy(
        idx_hbm.at[pl.ds(row0 * MAXLEN, ROWS_PER_W * MAXLEN)], idx_all)
    pltpu.sync_copy(pos_hbm, pos_v)

    def seed(buf):
      def sbody(i, carry):
        base = i * SEED_UNROLL
        for u in range(SEED_UNROLL):
          for t in range(DIM // LANES):
            buf[base + u, pl.ds(t * LANES, LANES)] = (
                pos_v[base + u, pl.ds(t * LANES, LANES)])
        return carry
      lax.fori_loop(0, MAXLEN // SEED_UNROLL, sbody, 0)

    def fire(p, r):
      return [
          pltpu.async_copy(
              tok_hbm.at[idx_all.at[pl.ds(r * MAXLEN + o, n)]],
              bufs[p].at[pl.ds(o, n)],
              gsem[p],
              add=True,
          )
          for o, n in zip(IDX_OFFS, IDX_CHUNKS)
      ]

    def wait_gathers(p, r):
      for o, n in zip(IDX_OFFS, IDX_CHUNKS):
        pltpu.make_async_copy(
            tok_hbm.at[idx_all.at[pl.ds(r * MAXLEN + o, n)]],
            bufs[p].at[pl.ds(o, n)],
            gsem[p],
        ).wait()

    def fire_out(p, r):
      return pltpu.async_copy(
          bufs[p], out_hbm.at[row0 + r].at[:, pl.ds(0, DIM)], osem[p])

    def wait_out(p, r):
      pltpu.make_async_copy(
          bufs[p], out_hbm.at[row0 + r].at[:, pl.ds(0, DIM)], osem[p]).wait()

    for p in range(NBUF):
      seed(bufs[p])
      fire(p, p)

    def body(g, carry):
      for p in range(NBUF):
        r = NBUF * g + p
        wait_gathers(p, r)
        fire_out(p, r)
        nr = r + NBUF

        @pl.when(nr < ROWS_PER_W)
        def _():
          wait_out(p, r)
          seed(bufs[p])
          fire(p, nr)
      return carry

    lax.fori_loop(0, ROWS_PER_W // NBUF, body, 0)
    for p in range(NBUF):
      wait_out(p, ROWS_PER_W - NBUF + p)

  return tok_pos_embed


_KERNEL = _make_kernel()

_TCHUNK = 2048
_TGRID = (VOCAB + _TCHUNK - 1) // _TCHUNK
_VPAD = _TGRID * _TCHUNK


def _transpose_body(in_ref, out_ref):
  x = in_ref[...]
  h = _TCHUNK // 2
  out_ref[...] = jnp.concatenate([x[:, :h].T, x[:, h:].T], axis=1)


_TRANSPOSE = pl.pallas_call(
    _transpose_body,
    grid=(_TGRID,),
    in_specs=[pl.BlockSpec((DIM, _TCHUNK), lambda i: (0, i))],
    out_specs=pl.BlockSpec((_TCHUNK // 2, PDIM), lambda i: (i, 0)),
    out_shape=jax.ShapeDtypeStruct((_VPAD * DIM // PDIM, PDIM), jnp.float32),
)


def kernel(inputs, token_table, pos_table):
  v = inputs.astype(jnp.int32).reshape(BATCH * MAXLEN)
  r = v & (_TCHUNK - 1)
  idx = (v ^ r) + ((r & (_TCHUNK // 2 - 1)) << 1) + (r >> 10)
  tok = _TRANSPOSE(token_table.T).reshape(_VPAD, DIM)
  out = _KERNEL(idx, tok, pos_table)
  return out[:, :, :DIM]
